# baseline (speedup 1.0000x reference)
import functools

import jax
import jax.numpy as jnp
import numpy as np
from jax import lax
from jax.experimental import pallas as pl
from jax.experimental.pallas import tpu as pltpu
from jax.experimental.pallas import tpu_sc as plsc

HARD_NEG_MAX = 34.0
EASY_NEG_MAX = 15.0
POS_MAX = 18.0
LOGIT_THRESHOLD = float(np.log(np.float32(0.3)) - np.log(np.float32(0.7)))

B, H, W = 16, 384, 384
N = H * W
HALF = N // 2
CH = 2048
NCH = HALF // CH
STEPS = CH // 16
BIG = 1.0e9


def _sc_body(mask_h, pos_h, pred_h, out_h,
             m_v, p_v, pr_v, o_v, f_v, c_v, shared, dsem, fsem):
    c = lax.axis_index("c")
    s = lax.axis_index("s")
    batch = c * 8 + s // 2
    half_id = s % 2
    base = batch * N + half_id * HALF
    is_first = (half_id == 0)
    lane = lax.iota(jnp.int32, 16)
    lane_f = lane.astype(jnp.float32)

    def scan_half(h0, e0, q0):
        def chunk_body(carry):
            i, h, e, q = carry
            off = base + i * CH
            cp_m = pltpu.make_async_copy(mask_h.at[pl.ds(off, CH)], m_v, dsem)
            cp_p = pltpu.make_async_copy(pos_h.at[pl.ds(off, CH)], p_v, dsem)
            cp_r = pltpu.make_async_copy(pred_h.at[pl.ds(2 * off, 2 * CH)],
                                         pr_v, dsem)
            cp_m.start()
            cp_p.start()
            cp_r.start()
            cp_m.wait()
            cp_p.wait()
            cp_r.wait()

            def step(j, carry2):
                h, e, q = carry2
                mv = m_v[pl.ds(j * 16, 16)]
                pv = p_v[pl.ds(j * 16, 16)]
                idx = j * 32 + 2 * lane
                a0 = plsc.load_gather(pr_v, [idx])
                a1 = plsc.load_gather(pr_v, [idx + 1])
                score = jnp.where(a1 - a0 >= LOGIT_THRESHOLD, 1.0, 0.0)
                ng = mv * score
                cn = plsc.cumsum(ng) + h
                cm = plsc.cumsum(mv) + e
                cq = plsc.cumsum(pv) + q
                kh = jnp.where(cn <= HARD_NEG_MAX, ng, 0.0)
                ke = jnp.where(cm <= EASY_NEG_MAX, mv, 0.0)
                kq = jnp.where(cq <= POS_MAX, pv, 0.0)
                lab = jnp.where(kh + ke >= 1.0, 0.0, -1.0) + 2.0 * kq
                o_v[pl.ds(j * 16, 16)] = lab
                return jnp.max(cn), jnp.max(cm), jnp.max(cq)

            h, e, q = lax.fori_loop(0, STEPS, step, (h, e, q))
            pltpu.make_async_copy(o_v, out_h.at[pl.ds(off, CH)], dsem).start()
            pltpu.make_async_copy(o_v, out_h.at[pl.ds(off, CH)], dsem).wait()
            return i + 1, h, e, q

        def chunk_cond(carry):
            i, h, e, q = carry
            sat = ((h >= HARD_NEG_MAX) & (e >= EASY_NEG_MAX) & (q >= POS_MAX))
            return (i < NCH) & jnp.logical_not(sat)

        return lax.while_loop(chunk_cond, chunk_body,
                              (jnp.int32(0), h0, e0, q0))

    def init_fill(k, _):
        f_v[pl.ds(k * 16, 16)] = jnp.full((16,), -1.0, jnp.float32)
        return 0

    lax.fori_loop(0, STEPS, init_fill, 0)

    zero = jnp.float32(0.0)
    big = jnp.float32(BIG)
    init = jnp.where(is_first, zero, big)
    i_stop, h, e, q = scan_half(init, init, init)

    c_v[...] = (h * (lane_f == 0.0).astype(jnp.float32)
                + e * jnp.where(lane == 1, 1.0, 0.0)
                + q * jnp.where(lane == 2, 1.0, 0.0))
    pltpu.sync_copy(c_v, shared.at[s])

    def fire(i, _):
        pltpu.make_async_copy(f_v, out_h.at[pl.ds(base + i * CH, CH)],
                              fsem).start()
        return 0

    def drain(i, _):
        pltpu.make_async_copy(f_v, out_h.at[pl.ds(base + i * CH, CH)],
                              fsem).wait()
        return 0

    lax.fori_loop(i_stop, NCH, fire, 0)
    lax.fori_loop(i_stop, NCH, drain, 0)

    plsc.subcore_barrier()

    partner = s - half_id
    pltpu.sync_copy(shared.at[partner], c_v)
    v = c_v[...]
    h0 = jnp.sum(jnp.where(lane == 0, v, 0.0))
    e0 = jnp.sum(jnp.where(lane == 1, v, 0.0))
    q0 = jnp.sum(jnp.where(lane == 2, v, 0.0))
    h0 = jnp.where(is_first, big, h0)
    e0 = jnp.where(is_first, big, e0)
    q0 = jnp.where(is_first, big, q0)
    scan_half(h0, e0, q0)


@functools.partial(
    pl.kernel,
    out_type=jax.ShapeDtypeStruct((B * N,), jnp.float32),
    mesh=plsc.VectorSubcoreMesh(core_axis_name="c", subcore_axis_name="s"),
    compiler_params=pltpu.CompilerParams(needs_layout_passes=False),
    scratch_types=[
        pltpu.VMEM((CH,), jnp.float32),
        pltpu.VMEM((CH,), jnp.float32),
        pltpu.VMEM((2 * CH,), jnp.float32),
        pltpu.VMEM((CH,), jnp.float32),
        pltpu.VMEM((CH,), jnp.float32),
        pltpu.VMEM((16,), jnp.float32),
        pltpu.VMEM_SHARED((16, 16), jnp.float32),
        pltpu.SemaphoreType.DMA,
        pltpu.SemaphoreType.DMA,
    ],
)
def _label_update_sc(mask_h, pos_h, pred_h, out_h, *scratch):
    _sc_body(mask_h, pos_h, pred_h, out_h, *scratch)


@jax.jit
def kernel(mask, pos_label, pred):
    out = _label_update_sc(
        mask.reshape(B * N),
        pos_label.reshape(B * N),
        pred.reshape(B * N * 2),
    )
    return out.reshape(B, H, W)


# speedup vs baseline: 39.8114x; 39.8114x over previous
import functools

import jax
import jax.numpy as jnp
import numpy as np
from jax import lax
from jax.experimental import pallas as pl
from jax.experimental.pallas import tpu as pltpu
from jax.experimental.pallas import tpu_sc as plsc

HARD_NEG_MAX = 34.0
EASY_NEG_MAX = 15.0
POS_MAX = 18.0
LOGIT_THRESHOLD = float(np.log(np.float32(0.3)) - np.log(np.float32(0.7)))

B, H, W = 16, 384, 384
HROWS = H // 2
R = 8
NCH = HROWS // R
CSTEPS = W // 16
FR = 96
NFILL = HROWS // FR
BIG = 1.0e9


def _sc_body(mask_h, pos_h, pred_h, out_h,
             m_v, p_v, pr_v, o_v, f_v, c_v, shared, dsem, fsem):
    c = lax.axis_index("c")
    s = lax.axis_index("s")
    batch = c * 8 + s // 2
    half_id = s % 2
    row0 = half_id * HROWS
    is_first = (half_id == 0)
    lane = lax.iota(jnp.int32, 16)

    def scan_half(h0, e0, q0):
        def chunk_body(carry):
            i, h, e, q = carry
            r = row0 + i * R
            cp_m = pltpu.make_async_copy(
                mask_h.at[batch, pl.ds(r, R), :], m_v, dsem)
            cp_p = pltpu.make_async_copy(
                pos_h.at[batch, pl.ds(r, R), :], p_v, dsem)
            cp_r = pltpu.make_async_copy(
                pred_h.at[batch, pl.ds(r * 6, R * 6), :], pr_v, dsem)
            cp_m.start()
            cp_p.start()
            cp_r.start()
            cp_m.wait()
            cp_p.wait()
            cp_r.wait()

            def row_step(rr, carry2):
                def col_step(cc, carry3):
                    h, e, q = carry3
                    mv = m_v[rr, pl.ds(cc * 16, 16)]
                    pv = p_v[rr, pl.ds(cc * 16, 16)]
                    wt = cc // 8
                    off = (cc % 8) * 16
                    a0 = pr_v[rr * 6 + wt * 2, pl.ds(off, 16)]
                    a1 = pr_v[rr * 6 + wt * 2 + 1, pl.ds(off, 16)]
                    score = jnp.where(a1 - a0 >= LOGIT_THRESHOLD, 1.0, 0.0)
                    ng = mv * score
                    cn = plsc.cumsum(ng) + h
                    cm = plsc.cumsum(mv) + e
                    cq = plsc.cumsum(pv) + q
                    kh = jnp.where(cn <= HARD_NEG_MAX, ng, 0.0)
                    ke = jnp.where(cm <= EASY_NEG_MAX, mv, 0.0)
                    kq = jnp.where(cq <= POS_MAX, pv, 0.0)
                    lab = jnp.where(kh + ke >= 1.0, 0.0, -1.0) + 2.0 * kq
                    o_v[rr, pl.ds(cc * 16, 16)] = lab
                    return jnp.max(cn), jnp.max(cm), jnp.max(cq)

                return lax.fori_loop(0, CSTEPS, col_step, carry2)

            h, e, q = lax.fori_loop(0, R, row_step, (h, e, q))
            cp_o = pltpu.make_async_copy(
                o_v, out_h.at[batch, pl.ds(r, R), :], dsem)
            cp_o.start()
            cp_o.wait()
            return i + 1, h, e, q

        def chunk_cond(carry):
            i, h, e, q = carry
            sat = ((h >= HARD_NEG_MAX) & (e >= EASY_NEG_MAX) & (q >= POS_MAX))
            return (i < NCH) & jnp.logical_not(sat)

        return lax.while_loop(chunk_cond, chunk_body,
                              (jnp.int32(0), h0, e0, q0))

    def init_fill(k, _):
        def init_row(cc, _):
            f_v[k, pl.ds(cc * 16, 16)] = jnp.full((16,), -1.0, jnp.float32)
            return 0

        lax.fori_loop(0, CSTEPS, init_row, 0)
        return 0

    lax.fori_loop(0, FR, init_fill, 0)

    for k in range(NFILL):
        pltpu.make_async_copy(
            f_v, out_h.at[batch, pl.ds(row0 + k * FR, FR), :], fsem).start()
    for k in range(NFILL):
        pltpu.make_async_copy(
            f_v, out_h.at[batch, pl.ds(row0 + k * FR, FR), :], fsem).wait()

    zero = jnp.float32(0.0)
    big = jnp.float32(BIG)
    init = jnp.where(is_first, zero, big)
    _, h, e, q = scan_half(init, init, init)

    c_v[...] = (h * jnp.where(lane == 0, 1.0, 0.0)
                + e * jnp.where(lane == 1, 1.0, 0.0)
                + q * jnp.where(lane == 2, 1.0, 0.0))
    pltpu.sync_copy(c_v, shared.at[s])

    plsc.subcore_barrier()

    partner = s - half_id
    pltpu.sync_copy(shared.at[partner], c_v)
    v = c_v[...]
    h0 = jnp.sum(jnp.where(lane == 0, v, 0.0))
    e0 = jnp.sum(jnp.where(lane == 1, v, 0.0))
    q0 = jnp.sum(jnp.where(lane == 2, v, 0.0))
    h0 = jnp.where(is_first, big, h0)
    e0 = jnp.where(is_first, big, e0)
    q0 = jnp.where(is_first, big, q0)
    scan_half(h0, e0, q0)


@functools.partial(
    pl.kernel,
    out_type=jax.ShapeDtypeStruct((B, H, W), jnp.float32),
    mesh=plsc.VectorSubcoreMesh(core_axis_name="c", subcore_axis_name="s"),
    compiler_params=pltpu.CompilerParams(needs_layout_passes=False),
    scratch_types=[
        pltpu.VMEM((R, W), jnp.float32),
        pltpu.VMEM((R, W), jnp.float32),
        pltpu.VMEM((R * 6, 128), jnp.float32),
        pltpu.VMEM((R, W), jnp.float32),
        pltpu.VMEM((FR, W), jnp.float32),
        pltpu.VMEM((16,), jnp.float32),
        pltpu.VMEM_SHARED((16, 16), jnp.float32),
        pltpu.SemaphoreType.DMA,
        pltpu.SemaphoreType.DMA,
    ],
)
def _label_update_sc(mask_h, pos_h, pred_h, out_h, *scratch):
    _sc_body(mask_h, pos_h, pred_h, out_h, *scratch)


@jax.jit
def kernel(mask, pos_label, pred):
    pred_view = (
        pred.reshape(B, H, 3, 128, 2)
        .transpose(0, 1, 2, 4, 3)
        .reshape(B, H * 6, 128)
    )
    return _label_update_sc(mask, pos_label, pred_view)


# speedup vs baseline: 40.8359x; 1.0257x over previous
import functools

import jax
import jax.numpy as jnp
import numpy as np
from jax import lax
from jax.experimental import pallas as pl
from jax.experimental.pallas import tpu as pltpu
from jax.experimental.pallas import tpu_sc as plsc

HARD_NEG_MAX = 34.0
EASY_NEG_MAX = 15.0
POS_MAX = 18.0
LOGIT_THRESHOLD = float(np.log(np.float32(0.3)) - np.log(np.float32(0.7)))

B, H, W = 16, 384, 384
HROWS = H // 2
R = 8
NCH = HROWS // R
CSTEPS = W // 16
FR = 96
NFILL = HROWS // FR
BIG = 1.0e9


def _sc_body(mask_h, pos_h, d_h, out_h,
             m_v, p_v, d_v, o_v, f_v, c_v, shared, dsem, fsem):
    c = lax.axis_index("c")
    s = lax.axis_index("s")
    batch = c * 8 + s // 2
    half_id = s % 2
    row0 = half_id * HROWS
    is_first = (half_id == 0)
    lane = lax.iota(jnp.int32, 16)

    def scan_half(h0, e0, q0):
        def chunk_body(carry):
            i, h, e, q = carry
            r = row0 + i * R
            cp_m = pltpu.make_async_copy(
                mask_h.at[batch, pl.ds(r, R), :], m_v, dsem)
            cp_p = pltpu.make_async_copy(
                pos_h.at[batch, pl.ds(r, R), :], p_v, dsem)
            cp_r = pltpu.make_async_copy(
                d_h.at[batch, pl.ds(r, R), :], d_v, dsem)
            cp_m.start()
            cp_p.start()
            cp_r.start()
            cp_m.wait()
            cp_p.wait()
            cp_r.wait()

            def row_step(rr, carry2):
                carry3 = carry2
                for cc in range(CSTEPS):
                    h, e, q = carry3
                    mv = m_v[rr, pl.ds(cc * 16, 16)]
                    pv = p_v[rr, pl.ds(cc * 16, 16)]
                    dv = d_v[rr, pl.ds(cc * 16, 16)]
                    score = jnp.where(dv >= LOGIT_THRESHOLD, 1.0, 0.0)
                    ng = mv * score
                    cn = plsc.cumsum(ng) + h
                    cm = plsc.cumsum(mv) + e
                    cq = plsc.cumsum(pv) + q
                    kh = jnp.where(cn <= HARD_NEG_MAX, ng, 0.0)
                    ke = jnp.where(cm <= EASY_NEG_MAX, mv, 0.0)
                    kq = jnp.where(cq <= POS_MAX, pv, 0.0)
                    lab = jnp.where(kh + ke >= 1.0, 0.0, -1.0) + 2.0 * kq
                    o_v[rr, pl.ds(cc * 16, 16)] = lab
                    carry3 = (jnp.max(cn), jnp.max(cm), jnp.max(cq))
                return carry3

            h, e, q = lax.fori_loop(0, R, row_step, (h, e, q))
            cp_o = pltpu.make_async_copy(
                o_v, out_h.at[batch, pl.ds(r, R), :], dsem)
            cp_o.start()
            cp_o.wait()
            return i + 1, h, e, q

        def chunk_cond(carry):
            i, h, e, q = carry
            sat = ((h >= HARD_NEG_MAX) & (e >= EASY_NEG_MAX) & (q >= POS_MAX))
            return (i < NCH) & jnp.logical_not(sat)

        return lax.while_loop(chunk_cond, chunk_body,
                              (jnp.int32(0), h0, e0, q0))

    def init_fill(k, _):
        def init_row(cc, _):
            f_v[k, pl.ds(cc * 16, 16)] = jnp.full((16,), -1.0, jnp.float32)
            return 0

        lax.fori_loop(0, CSTEPS, init_row, 0)
        return 0

    lax.fori_loop(0, FR, init_fill, 0)

    for k in range(NFILL):
        pltpu.make_async_copy(
            f_v, out_h.at[batch, pl.ds(row0 + k * FR, FR), :], fsem).start()
    for k in range(NFILL):
        pltpu.make_async_copy(
            f_v, out_h.at[batch, pl.ds(row0 + k * FR, FR), :], fsem).wait()

    zero = jnp.float32(0.0)
    big = jnp.float32(BIG)
    init = jnp.where(is_first, zero, big)
    _, h, e, q = scan_half(init, init, init)

    c_v[...] = (h * jnp.where(lane == 0, 1.0, 0.0)
                + e * jnp.where(lane == 1, 1.0, 0.0)
                + q * jnp.where(lane == 2, 1.0, 0.0))
    pltpu.sync_copy(c_v, shared.at[s])

    plsc.subcore_barrier()

    partner = s - half_id
    pltpu.sync_copy(shared.at[partner], c_v)
    v = c_v[...]
    h0 = jnp.sum(jnp.where(lane == 0, v, 0.0))
    e0 = jnp.sum(jnp.where(lane == 1, v, 0.0))
    q0 = jnp.sum(jnp.where(lane == 2, v, 0.0))
    h0 = jnp.where(is_first, big, h0)
    e0 = jnp.where(is_first, big, e0)
    q0 = jnp.where(is_first, big, q0)
    scan_half(h0, e0, q0)


@functools.partial(
    pl.kernel,
    out_type=jax.ShapeDtypeStruct((B, H, W), jnp.float32),
    mesh=plsc.VectorSubcoreMesh(core_axis_name="c", subcore_axis_name="s"),
    compiler_params=pltpu.CompilerParams(needs_layout_passes=False),
    scratch_types=[
        pltpu.VMEM((R, W), jnp.float32),
        pltpu.VMEM((R, W), jnp.float32),
        pltpu.VMEM((R, W), jnp.float32),
        pltpu.VMEM((R, W), jnp.float32),
        pltpu.VMEM((FR, W), jnp.float32),
        pltpu.VMEM((16,), jnp.float32),
        pltpu.VMEM_SHARED((16, 16), jnp.float32),
        pltpu.SemaphoreType.DMA,
        pltpu.SemaphoreType.DMA,
    ],
)
def _label_update_sc(mask_h, pos_h, pred_h, out_h, *scratch):
    _sc_body(mask_h, pos_h, pred_h, out_h, *scratch)


@jax.jit
def kernel(mask, pos_label, pred):
    d = pred[..., 1] - pred[..., 0]
    return _label_update_sc(mask, pos_label, d)
